# baseline (speedup 1.0000x reference)
import jax
import jax.numpy as jnp
from jax import lax
from jax.experimental import pallas as pl
from jax.experimental.pallas import tpu as pltpu
from jax.experimental.pallas import tpu_sc as plsc

N_ATOM_BASIS = 128
VOCAB = 100
NUM_ATOMS = 100000

_BLK = 128
_NW = 32
_CHUNK = 3128
_LAST_CHUNK_START = NUM_ATOMS - _CHUNK
_ITERS = 25
_LAST_OFF = _CHUNK - _BLK
_NBUF = 6
_DEPTH = 3


def _gather_kernel(table_hbm, idx_hbm, out_hbm, table_sh, idx_v, *bufs):
    rows = bufs[:_NBUF]
    gsem = bufs[_NBUF:2 * _NBUF]
    ssem = bufs[2 * _NBUF:3 * _NBUF]
    sid = lax.axis_index("s")
    wid = sid * 2 + lax.axis_index("c")
    chunk = jnp.minimum(wid * _CHUNK, _LAST_CHUNK_START)

    @pl.when(sid == 0)
    def _stage_table():
        pltpu.sync_copy(table_hbm, table_sh)

    pltpu.sync_copy(idx_hbm.at[pl.ds(chunk, _CHUNK)], idx_v)
    plsc.subcore_barrier()

    gathers, stores = [], []
    for step in range(_ITERS + _DEPTH):
        if step < _ITERS:
            i = step
            b = i % _NBUF
            off = min(i * _BLK, _LAST_OFF)
            if i >= _NBUF:
                stores[i - _NBUF].wait()
            gathers.append(
                pltpu.async_copy(
                    table_sh.at[idx_v.at[pl.ds(off, _BLK)]], rows[b], gsem[b]
                )
            )
        if step >= _DEPTH:
            k = step - _DEPTH
            b = k % _NBUF
            off = min(k * _BLK, _LAST_OFF)
            gathers[k].wait()
            stores.append(
                pltpu.async_copy(
                    rows[b], out_hbm.at[pl.ds(chunk + off, _BLK)], ssem[b]
                )
            )
    for k in range(max(0, _ITERS - _NBUF), _ITERS):
        stores[k].wait()


_ZROWS = 8192
_ZCHUNKS = 13
_ZLAST = NUM_ATOMS - _ZROWS
_ZSEMS = 8


def _zeros_body(o_ref, zb, sems):
    zb[...] = jnp.zeros_like(zb)
    copies = []
    for p in range(3):
        for j in range(_ZCHUNKS):
            r = min(j * _ZROWS, _ZLAST)
            k = (p * _ZCHUNKS + j) % _ZSEMS
            cp = pltpu.make_async_copy(
                zb, o_ref.at[p, pl.ds(r, _ZROWS), :], sems.at[k]
            )
            cp.start()
            copies.append(cp)
    for cp in copies:
        cp.wait()


@jax.jit
def kernel(z_number, atom_embed_weight):
    z = z_number.astype(jnp.int32)

    mesh = plsc.VectorSubcoreMesh(core_axis_name="c", subcore_axis_name="s")
    gather = pl.kernel(
        _gather_kernel,
        mesh=mesh,
        out_type=jax.ShapeDtypeStruct((NUM_ATOMS, N_ATOM_BASIS), jnp.float32),
        scratch_types=(
            [pltpu.VMEM_SHARED((VOCAB, N_ATOM_BASIS), jnp.float32)]
            + [pltpu.VMEM((_CHUNK,), jnp.int32)]
            + [pltpu.VMEM((_BLK, N_ATOM_BASIS), jnp.float32)] * _NBUF
            + [pltpu.SemaphoreType.DMA] * (2 * _NBUF)
        ),
    )
    s_i = gather(atom_embed_weight, z)

    v_planes = pl.pallas_call(
        _zeros_body,
        out_shape=jax.ShapeDtypeStruct((3, NUM_ATOMS, N_ATOM_BASIS), jnp.float32),
        out_specs=pl.BlockSpec(memory_space=pl.ANY),
        scratch_shapes=[
            pltpu.VMEM((_ZROWS, N_ATOM_BASIS), jnp.float32),
            pltpu.SemaphoreType.DMA((_ZSEMS,)),
        ],
    )()
    v_i = jnp.transpose(v_planes, (1, 2, 0))
    return (s_i, v_i)


# speedup vs baseline: 1.0256x; 1.0256x over previous
import jax
import jax.numpy as jnp
from jax import lax
from jax.experimental import pallas as pl
from jax.experimental.pallas import tpu as pltpu
from jax.experimental.pallas import tpu_sc as plsc

N_ATOM_BASIS = 128
VOCAB = 100
NUM_ATOMS = 100000

_BLK = 128
_NW = 32
_CHUNK = 3128
_LAST_CHUNK_START = NUM_ATOMS - _CHUNK
_ITERS = 25
_LAST_OFF = _CHUNK - _BLK
_NBUF = 6
_DEPTH = 3


def _gather_kernel(table_hbm, idx_hbm, out_hbm, table_sh, idx_v, *bufs):
    rows = bufs[:_NBUF]
    gsem = bufs[_NBUF:2 * _NBUF]
    ssem = bufs[2 * _NBUF:3 * _NBUF]
    sid = lax.axis_index("s")
    wid = sid * 2 + lax.axis_index("c")
    chunk = jnp.minimum(wid * _CHUNK, _LAST_CHUNK_START)

    @pl.when(sid == 0)
    def _stage_table():
        pltpu.sync_copy(table_hbm, table_sh)

    pltpu.sync_copy(idx_hbm.at[pl.ds(chunk, _CHUNK)], idx_v)
    plsc.subcore_barrier()

    gathers, stores = [], []
    for step in range(_ITERS + _DEPTH):
        if step < _ITERS:
            i = step
            b = i % _NBUF
            off = min(i * _BLK, _LAST_OFF)
            if i >= _NBUF:
                stores[i - _NBUF].wait()
            gathers.append(
                pltpu.async_copy(
                    table_sh.at[idx_v.at[pl.ds(off, _BLK)]], rows[b], gsem[b]
                )
            )
        if step >= _DEPTH:
            k = step - _DEPTH
            b = k % _NBUF
            off = min(k * _BLK, _LAST_OFF)
            gathers[k].wait()
            stores.append(
                pltpu.async_copy(
                    rows[b], out_hbm.at[pl.ds(chunk + off, _BLK)], ssem[b]
                )
            )
    for k in range(max(0, _ITERS - _NBUF), _ITERS):
        stores[k].wait()


def _zeros_body(o_ref):
    o_ref[...] = jnp.zeros_like(o_ref)


@jax.jit
def kernel(z_number, atom_embed_weight):
    z = z_number.astype(jnp.int32)

    mesh = plsc.VectorSubcoreMesh(core_axis_name="c", subcore_axis_name="s")
    gather = pl.kernel(
        _gather_kernel,
        mesh=mesh,
        out_type=jax.ShapeDtypeStruct((NUM_ATOMS, N_ATOM_BASIS), jnp.float32),
        scratch_types=(
            [pltpu.VMEM_SHARED((VOCAB, N_ATOM_BASIS), jnp.float32)]
            + [pltpu.VMEM((_CHUNK,), jnp.int32)]
            + [pltpu.VMEM((_BLK, N_ATOM_BASIS), jnp.float32)] * _NBUF
            + [pltpu.SemaphoreType.DMA] * (2 * _NBUF)
        ),
    )
    s_i = gather(atom_embed_weight, z)

    v_planes = pl.pallas_call(
        _zeros_body,
        out_shape=jax.ShapeDtypeStruct((3, NUM_ATOMS, N_ATOM_BASIS), jnp.float32),
        grid=(10,),
        out_specs=pl.BlockSpec((3, 10000, N_ATOM_BASIS), lambda i: (0, i, 0)),
    )()
    v_i = jnp.transpose(v_planes, (1, 2, 0))
    return (s_i, v_i)


# speedup vs baseline: 1.0325x; 1.0068x over previous
import jax
import jax.numpy as jnp
from jax import lax
from jax.experimental import pallas as pl
from jax.experimental.pallas import tpu as pltpu
from jax.experimental.pallas import tpu_sc as plsc

N_ATOM_BASIS = 128
VOCAB = 100
NUM_ATOMS = 100000

_BLK = 128
_NW = 32
_CHUNK = 3128
_LAST_CHUNK_START = NUM_ATOMS - _CHUNK
_ITERS = 25
_LAST_OFF = _CHUNK - _BLK
_NBUF = 5
_GROUPS = 5


def _gather_kernel(table_hbm, idx_hbm, out_hbm, table_sh, idx_v, *bufs):
    rows = bufs[:_NBUF]
    gsem = bufs[_NBUF:2 * _NBUF]
    ssem = bufs[2 * _NBUF:3 * _NBUF]
    sid = lax.axis_index("s")
    wid = sid * 2 + lax.axis_index("c")
    chunk = jnp.minimum(wid * _CHUNK, _LAST_CHUNK_START)

    @pl.when(sid == 0)
    def _stage_table():
        pltpu.sync_copy(table_hbm, table_sh)

    pltpu.sync_copy(idx_hbm.at[pl.ds(chunk, _CHUNK)], idx_v)
    plsc.subcore_barrier()

    def group(j, _):
        offs = [jnp.minimum((j * _NBUF + b) * _BLK, _LAST_OFF)
                for b in range(_NBUF)]
        hs = [
            pltpu.async_copy(
                table_sh.at[idx_v.at[pl.ds(offs[b], _BLK)]], rows[b], gsem[b]
            )
            for b in range(_NBUF)
        ]
        ss = []
        for b in range(_NBUF):
            hs[b].wait()
            ss.append(
                pltpu.async_copy(
                    rows[b], out_hbm.at[pl.ds(chunk + offs[b], _BLK)], ssem[b]
                )
            )
        for st in ss:
            st.wait()
        return ()

    lax.fori_loop(0, _GROUPS, group, ())


def _zeros_body(o_ref):
    o_ref[...] = jnp.zeros_like(o_ref)


@jax.jit
def kernel(z_number, atom_embed_weight):
    z = z_number.astype(jnp.int32)

    mesh = plsc.VectorSubcoreMesh(core_axis_name="c", subcore_axis_name="s")
    gather = pl.kernel(
        _gather_kernel,
        mesh=mesh,
        out_type=jax.ShapeDtypeStruct((NUM_ATOMS, N_ATOM_BASIS), jnp.float32),
        scratch_types=(
            [pltpu.VMEM_SHARED((VOCAB, N_ATOM_BASIS), jnp.float32)]
            + [pltpu.VMEM((_CHUNK,), jnp.int32)]
            + [pltpu.VMEM((_BLK, N_ATOM_BASIS), jnp.float32)] * _NBUF
            + [pltpu.SemaphoreType.DMA] * (2 * _NBUF)
        ),
    )
    s_i = gather(atom_embed_weight, z)

    v_planes = pl.pallas_call(
        _zeros_body,
        out_shape=jax.ShapeDtypeStruct((3, NUM_ATOMS, N_ATOM_BASIS), jnp.float32),
        grid=(20,),
        out_specs=pl.BlockSpec((3, 5000, N_ATOM_BASIS), lambda i: (0, i, 0)),
    )()
    v_i = jnp.transpose(v_planes, (1, 2, 0))
    return (s_i, v_i)
